# Initial kernel scaffold; baseline (speedup 1.0000x reference)
import jax
import jax.numpy as jnp
from jax.experimental import pallas as pl


def kernel(prot_data, compound_data, train_prot_avg_emb, train_compound_avg_emb, W_p_q, b_p_q, W_m_q, b_m_q, W_p_k, b_p_k, W_m_k, b_m_k, W_p_v, b_p_v, W_m_v, b_m_v, W_p_out1, b_p_out1, W_m_out1, b_m_out1, W_p_out2, b_p_out2, W_m_out2, b_m_out2):
    raise NotImplementedError("write your pallas kernel here")


# speedup vs baseline: 24.4086x; 24.4086x over previous
import math

import jax
import jax.numpy as jnp
from jax import lax
from jax.experimental import pallas as pl

H = 128
B = 64
KROWS = 100000
TOPK = 64
NCHUNK = 25
CHUNK = KROWS // NCHUNK
NSLICE = CHUNK // H
TAIL = CHUNK - NSLICE * H
R1 = 5
CAND = R1 * H
R2 = 10
NEG = -3.0e38
INV_SQRT_H = 1.0 / math.sqrt(H)


def _qk(q_ref, wq_ref, bq_ref, wk_ref):
    qk = jnp.dot(q_ref[...], wq_ref[...].T, preferred_element_type=jnp.float32)
    qk = qk + bq_ref[...]
    return jnp.dot(qk, wk_ref[...], preferred_element_type=jnp.float32)


def _insert(regs, blk):
    out = []
    for r in regs:
        hi = jnp.maximum(r, blk)
        blk = jnp.minimum(r, blk)
        out.append(hi)
    return out


def _cand_kernel(q_ref, wq_ref, bq_ref, wk_ref, tab_ref, cand_ref):
    qk = _qk(q_ref, wq_ref, bq_ref, wk_ref)
    sim = jnp.dot(qk, tab_ref[...].T, preferred_element_type=jnp.float32)
    regs = [jnp.full((B, H), NEG, jnp.float32) for _ in range(R1)]
    for s in range(NSLICE):
        regs = _insert(regs, sim[:, s * H:(s + 1) * H])
    tail = jnp.concatenate(
        [sim[:, NSLICE * H:CHUNK], jnp.full((B, H - TAIL), NEG, jnp.float32)], axis=1)
    regs = _insert(regs, tail)
    cand_ref[...] = jnp.concatenate(regs, axis=1)


def _candidates(query, Wq, bq, Wk, table):
    return pl.pallas_call(
        _cand_kernel,
        grid=(NCHUNK,),
        in_specs=[
            pl.BlockSpec((B, H), lambda i: (0, 0)),
            pl.BlockSpec((H, H), lambda i: (0, 0)),
            pl.BlockSpec((1, H), lambda i: (0, 0)),
            pl.BlockSpec((H, H), lambda i: (0, 0)),
            pl.BlockSpec((CHUNK, H), lambda i: (i, 0)),
        ],
        out_specs=pl.BlockSpec((B, CAND), lambda i: (0, i)),
        out_shape=jax.ShapeDtypeStruct((B, NCHUNK * CAND), jnp.float32),
    )(query, Wq, bq.reshape(1, H), Wk, table)


def _thresh_kernel(cand_ref, tm_ref, ms_ref):
    ncand = NCHUNK * CAND
    nsl = ncand // H

    def red_body(s, regs):
        blk = cand_ref[:, pl.ds(s * H, H)]
        return tuple(_insert(list(regs), blk))

    regs0 = tuple(jnp.full((B, H), NEG, jnp.float32) for _ in range(R2))
    regs = lax.fori_loop(0, nsl, red_body, regs0)
    m = jnp.concatenate(list(regs), axis=1)
    slot = lax.broadcasted_iota(jnp.int32, (B, R2 * H), 1)
    big = jnp.int32(R2 * H)

    def ext_body(r, carry):
        m, mstar, v64, v65 = carry
        v = jnp.max(m, axis=1, keepdims=True)
        mstar = jnp.where(r == 0, v, mstar)
        v64 = jnp.where(r == TOPK - 1, v, v64)
        v65 = jnp.where(r == TOPK, v, v65)
        pos = jnp.min(jnp.where(m == v, slot, big), axis=1, keepdims=True)
        m = jnp.where(slot == pos, NEG, m)
        return m, mstar, v64, v65

    z = jnp.zeros((B, 1), jnp.float32)
    _, mstar, v64, v65 = lax.fori_loop(0, TOPK + 1, ext_body, (m, z, z, z))
    tm_ref[...] = jnp.broadcast_to((v64 + v65) * 0.5, (B, H))
    ms_ref[...] = jnp.broadcast_to(mstar, (B, H))


def _threshold(cand):
    return pl.pallas_call(
        _thresh_kernel,
        in_specs=[pl.BlockSpec((B, NCHUNK * CAND), lambda: (0, 0))],
        out_specs=[pl.BlockSpec((B, H), lambda: (0, 0))] * 2,
        out_shape=[jax.ShapeDtypeStruct((B, H), jnp.float32)] * 2,
    )(cand)


def _acc_kernel(q_ref, wq_ref, bq_ref, wk_ref, tab_ref, tm_ref, ms_ref,
                acc_ref, z_ref):
    @pl.when(pl.program_id(0) == 0)
    def _():
        acc_ref[...] = jnp.zeros_like(acc_ref)
        z_ref[...] = jnp.zeros_like(z_ref)

    qk = _qk(q_ref, wq_ref, bq_ref, wk_ref)
    sim = jnp.dot(qk, tab_ref[...].T, preferred_element_type=jnp.float32)
    t = tm_ref[...][:, 0:1]
    ms = ms_ref[...][:, 0:1]
    p = jnp.where(sim >= t, jnp.exp((sim - ms) * INV_SQRT_H), 0.0)
    acc_ref[...] += jnp.dot(p, tab_ref[...], preferred_element_type=jnp.float32)
    z_ref[...] += jnp.broadcast_to(jnp.sum(p, axis=1, keepdims=True), (B, H))


def _accumulate(query, Wq, bq, Wk, table, tmid, mstar):
    return pl.pallas_call(
        _acc_kernel,
        grid=(NCHUNK,),
        in_specs=[
            pl.BlockSpec((B, H), lambda i: (0, 0)),
            pl.BlockSpec((H, H), lambda i: (0, 0)),
            pl.BlockSpec((1, H), lambda i: (0, 0)),
            pl.BlockSpec((H, H), lambda i: (0, 0)),
            pl.BlockSpec((CHUNK, H), lambda i: (i, 0)),
            pl.BlockSpec((B, H), lambda i: (0, 0)),
            pl.BlockSpec((B, H), lambda i: (0, 0)),
        ],
        out_specs=[pl.BlockSpec((B, H), lambda i: (0, 0))] * 2,
        out_shape=[jax.ShapeDtypeStruct((B, H), jnp.float32)] * 2,
    )(query, Wq, bq.reshape(1, H), Wk, table, tmid, mstar)


def _head_kernel(p_ref, c_ref, apm_ref, zpm_ref, amp_ref, zmp_ref,
                 wpv_ref, bpv_ref, wmv_ref, bmv_ref,
                 wpo1_ref, bpo1_ref, wpo2_ref, bpo2_ref,
                 wmo1_ref, bmo1_ref, wmo2_ref, bmo2_ref, out_ref):
    s_pm = apm_ref[...] / zpm_ref[...]
    s_mp = amp_ref[...] / zmp_ref[...]
    sim_pm = jnp.dot(s_pm, wmv_ref[...].T, preferred_element_type=jnp.float32) + bmv_ref[...]
    sim_mp = jnp.dot(s_mp, wpv_ref[...].T, preferred_element_type=jnp.float32) + bpv_ref[...]
    prot2 = jnp.concatenate([p_ref[...], sim_pm], axis=-1)
    comp2 = jnp.concatenate([c_ref[...], sim_mp], axis=-1)
    prot2 = jnp.dot(prot2, wpo1_ref[...].T, preferred_element_type=jnp.float32) + bpo1_ref[...]
    prot2 = jnp.maximum(prot2, 0.0)
    prot2 = jnp.dot(prot2, wpo2_ref[...].T, preferred_element_type=jnp.float32) + bpo2_ref[...]
    comp2 = jnp.dot(comp2, wmo1_ref[...].T, preferred_element_type=jnp.float32) + bmo1_ref[...]
    comp2 = jnp.maximum(comp2, 0.0)
    comp2 = jnp.dot(comp2, wmo2_ref[...].T, preferred_element_type=jnp.float32) + bmo2_ref[...]
    pn = prot2 * lax.rsqrt(jnp.maximum(jnp.sum(prot2 * prot2, axis=1, keepdims=True), 1e-24))
    mn = comp2 * lax.rsqrt(jnp.maximum(jnp.sum(comp2 * comp2, axis=1, keepdims=True), 1e-24))
    out_ref[...] = (jnp.sum(pn * mn, axis=1, keepdims=True).T + 1.0) * 0.5


def _head(prot_data, compound_data, acc_pm, z_pm, acc_mp, z_mp,
          W_p_v, b_p_v, W_m_v, b_m_v,
          W_p_out1, b_p_out1, W_p_out2, b_p_out2,
          W_m_out1, b_m_out1, W_m_out2, b_m_out2):
    args = [prot_data, compound_data, acc_pm, z_pm, acc_mp, z_mp,
            W_p_v, b_p_v.reshape(1, H), W_m_v, b_m_v.reshape(1, H),
            W_p_out1, b_p_out1.reshape(1, H), W_p_out2, b_p_out2.reshape(1, H),
            W_m_out1, b_m_out1.reshape(1, H), W_m_out2, b_m_out2.reshape(1, H)]
    out = pl.pallas_call(
        _head_kernel,
        in_specs=[pl.BlockSpec(a.shape, lambda *_: (0,) * a.ndim) for a in args],
        out_specs=pl.BlockSpec((1, B), lambda: (0, 0)),
        out_shape=jax.ShapeDtypeStruct((1, B), jnp.float32),
    )(*args)
    return out.reshape(B)


def _direction(query, Wq, bq, Wk, table):
    cand = _candidates(query, Wq, bq, Wk, table)
    tmid, mstar = _threshold(cand)
    return _accumulate(query, Wq, bq, Wk, table, tmid, mstar)


def kernel(prot_data, compound_data, train_prot_avg_emb, train_compound_avg_emb,
           W_p_q, b_p_q, W_m_q, b_m_q, W_p_k, b_p_k, W_m_k, b_m_k,
           W_p_v, b_p_v, W_m_v, b_m_v,
           W_p_out1, b_p_out1, W_m_out1, b_m_out1,
           W_p_out2, b_p_out2, W_m_out2, b_m_out2):
    acc_mp, z_mp = _direction(compound_data, W_m_q, b_m_q, W_p_k, train_prot_avg_emb)
    acc_pm, z_pm = _direction(prot_data, W_p_q, b_p_q, W_m_k, train_compound_avg_emb)
    return _head(prot_data, compound_data, acc_pm, z_pm, acc_mp, z_mp,
                 W_p_v, b_p_v, W_m_v, b_m_v,
                 W_p_out1, b_p_out1, W_p_out2, b_p_out2,
                 W_m_out1, b_m_out1, W_m_out2, b_m_out2)
